# Initial kernel scaffold; baseline (speedup 1.0000x reference)
import jax
import jax.numpy as jnp
from jax.experimental import pallas as pl


def kernel(memory, source_nodes, timestamps, n_layers, W, b):
    raise NotImplementedError("write your pallas kernel here")


# speedup vs baseline: 1.0836x; 1.0836x over previous
import functools

import jax
import jax.numpy as jnp
from jax import lax
from jax.experimental import pallas as pl
from jax.experimental.pallas import tpu as pltpu
from jax.experimental.pallas import tpu_sc as plsc

D = 128
B = 16384
L = 16


def _make_sc_call():
    info = plsc.get_sparse_core_info()
    nc, ns = info.num_cores, info.num_subcores
    nw = nc * ns
    bpw = B // nw
    nch = bpw // 128
    mesh = plsc.VectorSubcoreMesh(core_axis_name="c", subcore_axis_name="s")

    @functools.partial(
        pl.kernel,
        mesh=mesh,
        out_type=jax.ShapeDtypeStruct((B, D), jnp.float32),
        scratch_types=[
            pltpu.VMEM((nch, 128), jnp.int32),
            pltpu.VMEM((bpw // L, L), jnp.float32),
            pltpu.VMEM((D,), jnp.float32),
            pltpu.VMEM((D,), jnp.float32),
            pltpu.VMEM((bpw, D), jnp.float32),
            pltpu.SemaphoreType.DMA,
        ],
    )
    def sc_kernel(mem_hbm, idx_hbm, t_hbm, w_hbm, b1_hbm, out_hbm,
                  idx_v, t_v, w_v, b1_v, rows_v, sem):
        wid = lax.axis_index("s") * nc + lax.axis_index("c")
        base = wid * bpw

        pltpu.sync_copy(idx_hbm.at[pl.ds(wid * nch, nch)], idx_v)
        pltpu.sync_copy(t_hbm.at[pl.ds(wid * (bpw // L), bpw // L)], t_v)
        pltpu.sync_copy(w_hbm, w_v)
        pltpu.sync_copy(b1_hbm, b1_v)

        copies = [
            pltpu.async_copy(mem_hbm.at[idx_v.at[c]],
                             rows_v.at[pl.ds(c * 128, 128)], sem)
            for c in range(nch)
        ]
        for cp in copies:
            cp.wait()

        wl = [w_v[pl.ds(c * L, L)] for c in range(D // L)]
        b1l = [b1_v[pl.ds(c * L, L)] for c in range(D // L)]

        def group_body(g, _):
            tvec = t_v.at[g][...]
            for r in range(L):
                t16 = jnp.broadcast_to(tvec[r], (L,))
                row = rows_v.at[g * L + r]
                for c in range(D // L):
                    sl = pl.ds(c * L, L)
                    row[sl] = row[sl] * (t16 * wl[c] + b1l[c])
            return 0

        lax.fori_loop(0, bpw // L, group_body, 0)

        pltpu.sync_copy(rows_v, out_hbm.at[pl.ds(base, bpw)])

    return sc_kernel


def kernel(memory, source_nodes, timestamps, n_layers, W, b):
    del n_layers
    idx = source_nodes.astype(jnp.int32).reshape(B // 128, 128)
    w = W.reshape(D).astype(jnp.float32)
    b1 = (1.0 + b).astype(jnp.float32)
    ts = timestamps.astype(jnp.float32).reshape(B // L, L)
    sc = _make_sc_call()
    return sc(memory, idx, ts, w, b1)


# speedup vs baseline: 1.1639x; 1.0741x over previous
import functools

import jax
import jax.numpy as jnp
from jax import lax
from jax.experimental import pallas as pl
from jax.experimental.pallas import tpu as pltpu
from jax.experimental.pallas import tpu_sc as plsc

D = 128
B = 16384
L = 16
CH = 128


def _make_sc_call():
    info = plsc.get_sparse_core_info()
    nc, ns = info.num_cores, info.num_subcores
    nw = nc * ns
    bpw = B // nw
    nch = bpw // CH
    mesh = plsc.VectorSubcoreMesh(core_axis_name="c", subcore_axis_name="s")

    @functools.partial(
        pl.kernel,
        mesh=mesh,
        out_type=jax.ShapeDtypeStruct((B, D), jnp.float32),
        scratch_types=[
            pltpu.VMEM((nch, CH), jnp.int32),
            pltpu.VMEM((bpw // L, L), jnp.float32),
            pltpu.VMEM((D,), jnp.float32),
            pltpu.VMEM((D,), jnp.float32),
            pltpu.VMEM((bpw, D), jnp.float32),
            pltpu.SemaphoreType.DMA,
            pltpu.SemaphoreType.DMA,
            pltpu.SemaphoreType.DMA,
            pltpu.SemaphoreType.DMA,
            pltpu.SemaphoreType.DMA,
            pltpu.SemaphoreType.DMA,
        ],
    )
    def sc_kernel(mem_hbm, idx_hbm, t_hbm, w_hbm, b_hbm, out_hbm,
                  idx_v, t_v, w_v, b_v, rows_v,
                  sem_meta, sem_g0, sem_g1, sem_g2, sem_g3, sem_out):
        sem_g = (sem_g0, sem_g1, sem_g2, sem_g3)
        wid = lax.axis_index("s") * nc + lax.axis_index("c")
        base = wid * bpw

        pltpu.sync_copy(idx_hbm.at[pl.ds(wid * nch, nch)], idx_v)

        gathers = [
            pltpu.async_copy(mem_hbm.at[idx_v.at[c]],
                             rows_v.at[pl.ds(c * CH, CH)], sem_g[c])
            for c in range(nch)
        ]

        t_copy = pltpu.async_copy(
            t_hbm.at[pl.ds(wid * (bpw // L), bpw // L)], t_v, sem_meta)
        w_copy = pltpu.async_copy(w_hbm, w_v, sem_meta)
        b_copy = pltpu.async_copy(b_hbm, b_v, sem_meta)
        t_copy.wait()
        w_copy.wait()
        b_copy.wait()

        one = jnp.full((L,), 1.0, jnp.float32)
        wl = [w_v[pl.ds(c * L, L)] for c in range(D // L)]
        b1l = [b_v[pl.ds(c * L, L)] + one for c in range(D // L)]

        gpc = CH // L
        out_copies = []
        for c in range(nch):
            gathers[c].wait()

            def group_body(g, _, c=c):
                tvec = t_v.at[c * gpc + g][...]
                for r in range(L):
                    t16 = jnp.broadcast_to(tvec[r], (L,))
                    row = rows_v.at[(c * gpc + g) * L + r]
                    for cc in range(D // L):
                        sl = pl.ds(cc * L, L)
                        row[sl] = row[sl] * (t16 * wl[cc] + b1l[cc])
                return 0

            lax.fori_loop(0, gpc, group_body, 0)
            out_copies.append(pltpu.async_copy(
                rows_v.at[pl.ds(c * CH, CH)],
                out_hbm.at[pl.ds(base + c * CH, CH)], sem_out))

        for cp in out_copies:
            cp.wait()

    return sc_kernel


def kernel(memory, source_nodes, timestamps, n_layers, W, b):
    del n_layers
    idx = source_nodes.astype(jnp.int32).reshape(B // CH, CH)
    ts = timestamps.astype(jnp.float32).reshape(B // L, L)
    w = W.reshape(D).astype(jnp.float32)
    sc = _make_sc_call()
    return sc(memory, idx, ts, w, b.astype(jnp.float32))
